# Initial kernel scaffold; baseline (speedup 1.0000x reference)
import functools

import jax
import jax.numpy as jnp
from jax import lax
from jax.experimental import pallas as pl
from jax.experimental.pallas import tpu as pltpu
from jax.experimental.pallas import tpu_sc as plsc

N = 10000
NP = 10240
E = 320000
NC, NS = 2, 16
NW = NC * NS
CHUNK = 128
CPW = 79
EPW = CPW * CHUNK
EP = NW * EPW
RPT = NP // NS


def _make_agg(D):
    mesh = plsc.VectorSubcoreMesh(
        core_axis_name="c", subcore_axis_name="s", num_cores=NC, num_subcores=NS
    )

    @functools.partial(
        pl.kernel,
        out_type=jax.ShapeDtypeStruct((NC, NP, D), jnp.float32),
        mesh=mesh,
        scratch_types=[
            pltpu.VMEM((CPW, CHUNK), jnp.int32),
            pltpu.VMEM((CPW, CHUNK), jnp.int32),
            pltpu.VMEM((CHUNK, D), jnp.float32),
            pltpu.VMEM_SHARED((NP, D), jnp.float32),
            pltpu.SemaphoreType.DMA,
        ],
    )
    def agg(table_hbm, src_hbm, dst_hbm, out_hbm, sidx, didx, rows, acc, sem):
        c = lax.axis_index("c")
        s = lax.axis_index("s")
        wid = s * NC + c
        pltpu.sync_copy(table_hbm.at[pl.ds(s * RPT, RPT)],
                        acc.at[pl.ds(s * RPT, RPT)])
        pltpu.sync_copy(src_hbm.at[wid], sidx)
        pltpu.sync_copy(dst_hbm.at[wid], didx)
        plsc.subcore_barrier()

        def body(i, _):
            pltpu.async_copy(table_hbm.at[sidx.at[i]], rows, sem).wait()
            pltpu.sync_copy(rows, acc.at[didx.at[i]], add=True)
            return ()

        lax.fori_loop(0, CPW, body, (), unroll=False)
        plsc.subcore_barrier()
        pltpu.sync_copy(acc.at[pl.ds(s * RPT, RPT)],
                        out_hbm.at[c, pl.ds(s * RPT, RPT)])

    return agg


_agg = {d: _make_agg(d) for d in (16, 32, 64)}


def _tc(fn, *args, out_shape):
    return pl.pallas_call(fn, out_shape=out_shape)(*args)


def _tc0_body(dp, xp, w1, dis_o, t1_o):
    dsum = dp[0] + dp[1] - 1.0
    dis = lax.rsqrt(dsum[:, 0:1])
    dis_o[...] = dis
    t1_o[...] = jnp.dot(xp[...], w1[...], preferred_element_type=jnp.float32) * dis


def _enc_body(u, t, dis, b, w, out):
    v = (u[0] + u[1] - t[...]) * dis[...] + b[...]
    h = jnp.maximum(v, 0.0)
    out[...] = jnp.dot(h, w[...], preferred_element_type=jnp.float32) * dis[...]


def _mid_body(u, t, dis, b, out):
    z = (u[0] + u[1] - t[...]) * dis[...] + b[...]
    out[...] = z * dis[...]


def _dec_body(u, t, dis, b, w, out):
    q = (u[0] + u[1] - t[...]) * dis[...]
    h = jnp.maximum(jnp.dot(q, w[...], preferred_element_type=jnp.float32) + b[...], 0.0)
    out[...] = h * dis[...]


def _fin_body(u, t, dis, b, w, out):
    q = (u[0] + u[1] - t[...]) * dis[...]
    out[...] = jnp.dot(q, w[...], preferred_element_type=jnp.float32) + b[...]


def kernel(x, edge_index, W1, b1, W2, b2, W3, b3, W4, b4, W5, b5, W6, b6):
    f32 = jnp.float32
    pad = EP - E
    src = jnp.concatenate([edge_index[0], jnp.zeros((pad,), jnp.int32)])
    dst = jnp.concatenate([edge_index[1], jnp.full((pad,), NP - 1, jnp.int32)])
    src = src.reshape(NW, CPW, CHUNK)
    dst = dst.reshape(NW, CPW, CHUNK)
    xp = jnp.concatenate([x, jnp.zeros((NP - N, x.shape[1]), f32)])
    ones16 = jnp.concatenate([jnp.ones((N, 16), f32), jnp.zeros((NP - N, 16), f32)])

    def agg(table):
        d = table.shape[-1]
        return _agg[d](table, src, dst)

    dp = agg(ones16)
    dis, t1 = _tc(
        _tc0_body, dp, xp, W1,
        out_shape=(jax.ShapeDtypeStruct((NP, 1), f32),
                   jax.ShapeDtypeStruct((NP, 64), f32)),
    )
    u1 = agg(t1)
    t2 = _tc(_enc_body, u1, t1, dis, b1, W2,
             out_shape=jax.ShapeDtypeStruct((NP, 32), f32))
    u2 = agg(t2)
    t3 = _tc(_enc_body, u2, t2, dis, b2, W3,
             out_shape=jax.ShapeDtypeStruct((NP, 16), f32))
    u3 = agg(t3)
    t4 = _tc(_mid_body, u3, t3, dis, b3,
             out_shape=jax.ShapeDtypeStruct((NP, 16), f32))
    u4 = agg(t4)
    t5 = _tc(_dec_body, u4, t4, dis, b4, W4,
             out_shape=jax.ShapeDtypeStruct((NP, 32), f32))
    u5 = agg(t5)
    t6 = _tc(_dec_body, u5, t5, dis, b5, W5,
             out_shape=jax.ShapeDtypeStruct((NP, 64), f32))
    u6 = agg(t6)
    outp = _tc(_fin_body, u6, t6, dis, b6, W6,
               out_shape=jax.ShapeDtypeStruct((NP, 128), f32))
    return outp[:N]


# speedup vs baseline: 16.1708x; 16.1708x over previous
import functools

import jax
import jax.numpy as jnp
from jax import lax
from jax.experimental import pallas as pl
from jax.experimental.pallas import tpu as pltpu
from jax.experimental.pallas import tpu_sc as plsc

N = 10000
NP = 10240
E = 320000
NC, NS = 2, 16
NW = NC * NS
CHUNK = 128
CPW = 79
EPW = CPW * CHUNK
EP = NW * EPW
RPT = NP // NS


def _make_agg(D):
    mesh = plsc.VectorSubcoreMesh(
        core_axis_name="c", subcore_axis_name="s", num_cores=NC, num_subcores=NS
    )

    @functools.partial(
        pl.kernel,
        out_type=jax.ShapeDtypeStruct((NC, NP, D), jnp.float32),
        mesh=mesh,
        scratch_types=[
            pltpu.VMEM((CPW, CHUNK), jnp.int32),
            pltpu.VMEM((CPW, CHUNK), jnp.int32),
            pltpu.VMEM((CHUNK, D), jnp.float32),
            pltpu.VMEM_SHARED((NP, D), jnp.float32),
            pltpu.SemaphoreType.DMA,
        ],
        compiler_params=pltpu.CompilerParams(use_tc_tiling_on_sc=False),
    )
    def agg(table_hbm, src_hbm, dst_hbm, out_hbm, sidx, didx, rows, acc, sem):
        c = lax.axis_index("c")
        s = lax.axis_index("s")
        wid = s * NC + c
        pltpu.sync_copy(table_hbm.at[pl.ds(s * RPT, RPT)],
                        acc.at[pl.ds(s * RPT, RPT)])
        pltpu.sync_copy(src_hbm.at[wid], sidx)
        pltpu.sync_copy(dst_hbm.at[wid], didx)
        plsc.subcore_barrier()

        def body(i, _):
            pltpu.async_copy(table_hbm.at[sidx.at[i]], rows, sem).wait()
            pltpu.sync_copy(rows, acc.at[didx.at[i]], add=True)
            return ()

        lax.fori_loop(0, CPW, body, (), unroll=False)
        plsc.subcore_barrier()
        pltpu.sync_copy(acc.at[pl.ds(s * RPT, RPT)],
                        out_hbm.at[c, pl.ds(s * RPT, RPT)])

    return agg


_agg = {d: _make_agg(d) for d in (16, 32, 64)}


def _tc(fn, *args, out_shape):
    return pl.pallas_call(fn, out_shape=out_shape)(*args)


def _tc0_body(dp, xp, w1, dis_o, t1_o):
    dsum = dp[0] + dp[1] - 1.0
    dis = lax.rsqrt(dsum[:, 0:1])
    dis_o[...] = dis
    t1_o[...] = jnp.dot(xp[...], w1[...], preferred_element_type=jnp.float32) * dis


def _enc_body(u, t, dis, b, w, out):
    v = (u[0] + u[1] - t[...]) * dis[...] + b[...]
    h = jnp.maximum(v, 0.0)
    out[...] = jnp.dot(h, w[...], preferred_element_type=jnp.float32) * dis[...]


def _mid_body(u, t, dis, b, out):
    z = (u[0] + u[1] - t[...]) * dis[...] + b[...]
    out[...] = z * dis[...]


def _dec_body(u, t, dis, b, w, out):
    q = (u[0] + u[1] - t[...]) * dis[...]
    h = jnp.maximum(jnp.dot(q, w[...], preferred_element_type=jnp.float32) + b[...], 0.0)
    out[...] = h * dis[...]


def _fin_body(u, t, dis, b, w, out):
    q = (u[0] + u[1] - t[...]) * dis[...]
    out[...] = jnp.dot(q, w[...], preferred_element_type=jnp.float32) + b[...]


def kernel(x, edge_index, W1, b1, W2, b2, W3, b3, W4, b4, W5, b5, W6, b6):
    f32 = jnp.float32
    pad = EP - E
    src = jnp.concatenate([edge_index[0], jnp.zeros((pad,), jnp.int32)])
    dst = jnp.concatenate([edge_index[1], jnp.full((pad,), NP - 1, jnp.int32)])
    src = src.reshape(NW, CPW, CHUNK)
    dst = dst.reshape(NW, CPW, CHUNK)
    xp = jnp.concatenate([x, jnp.zeros((NP - N, x.shape[1]), f32)])
    ones16 = jnp.concatenate([jnp.ones((N, 16), f32), jnp.zeros((NP - N, 16), f32)])
    b1, b2, b3 = b1.reshape(1, -1), b2.reshape(1, -1), b3.reshape(1, -1)
    b4, b5, b6 = b4.reshape(1, -1), b5.reshape(1, -1), b6.reshape(1, -1)

    def agg(table):
        d = table.shape[-1]
        return _agg[d](table, src, dst)

    dp = agg(ones16)
    dis, t1 = _tc(
        _tc0_body, dp, xp, W1,
        out_shape=(jax.ShapeDtypeStruct((NP, 1), f32),
                   jax.ShapeDtypeStruct((NP, 64), f32)),
    )
    u1 = agg(t1)
    t2 = _tc(_enc_body, u1, t1, dis, b1, W2,
             out_shape=jax.ShapeDtypeStruct((NP, 32), f32))
    u2 = agg(t2)
    t3 = _tc(_enc_body, u2, t2, dis, b2, W3,
             out_shape=jax.ShapeDtypeStruct((NP, 16), f32))
    u3 = agg(t3)
    t4 = _tc(_mid_body, u3, t3, dis, b3,
             out_shape=jax.ShapeDtypeStruct((NP, 16), f32))
    u4 = agg(t4)
    t5 = _tc(_dec_body, u4, t4, dis, b4, W4,
             out_shape=jax.ShapeDtypeStruct((NP, 32), f32))
    u5 = agg(t5)
    t6 = _tc(_dec_body, u5, t5, dis, b5, W5,
             out_shape=jax.ShapeDtypeStruct((NP, 64), f32))
    u6 = agg(t6)
    outp = _tc(_fin_body, u6, t6, dis, b6, W6,
               out_shape=jax.ShapeDtypeStruct((NP, 128), f32))
    return outp[:N]


# speedup vs baseline: 34.3523x; 2.1243x over previous
import functools

import jax
import jax.numpy as jnp
from jax import lax
from jax.experimental import pallas as pl
from jax.experimental.pallas import tpu as pltpu
from jax.experimental.pallas import tpu_sc as plsc

N = 10000
NP = 10240
E = 320000
NC, NS = 2, 16
NW = NC * NS
CHUNK = 128
CPW = 80
EPW = CPW * CHUNK
EP = NW * EPW
RPT = NP // NS


def _make_agg(D, constant_rows):
    mesh = plsc.VectorSubcoreMesh(
        core_axis_name="c", subcore_axis_name="s", num_cores=NC, num_subcores=NS
    )

    @functools.partial(
        pl.kernel,
        out_type=jax.ShapeDtypeStruct((NC, NP, D), jnp.float32),
        mesh=mesh,
        scratch_types=[
            pltpu.VMEM((CPW + 1, CHUNK), jnp.int32),
            pltpu.VMEM((CPW, CHUNK), jnp.int32),
            pltpu.VMEM((CHUNK, D), jnp.float32),
            pltpu.VMEM((CHUNK, D), jnp.float32),
            pltpu.VMEM_SHARED((NP, D), jnp.float32),
            pltpu.VMEM_SHARED((NP, D), jnp.float32),
            pltpu.SemaphoreType.DMA,
            pltpu.SemaphoreType.DMA,
            pltpu.SemaphoreType.DMA,
            pltpu.SemaphoreType.DMA,
        ],
        compiler_params=pltpu.CompilerParams(use_tc_tiling_on_sc=False),
    )
    def agg(table_hbm, src_hbm, dst_hbm, out_hbm,
            sidx, didx, rowsA, rowsB, acc, tbl, gA, gB, sA, sB):
        c = lax.axis_index("c")
        s = lax.axis_index("s")
        wid = s * NC + c
        sl = pl.ds(s * RPT, RPT)
        ini = pltpu.async_copy(table_hbm.at[sl], acc.at[sl], gA)
        pltpu.sync_copy(dst_hbm.at[wid], didx)
        if not constant_rows:
            pltpu.sync_copy(src_hbm.at[wid], sidx)
            pltpu.sync_copy(table_hbm.at[sl], tbl.at[sl])
        else:
            pltpu.sync_copy(table_hbm.at[pl.ds(0, CHUNK)], rowsA)
        ini.wait()
        plsc.subcore_barrier()

        if constant_rows:
            def body(i, _):
                pltpu.async_copy(rowsA, acc.at[didx.at[i]], sA, add=True).wait()
                return ()
            lax.fori_loop(0, CPW, body, (), unroll=False)
        else:
            def gather(i, buf, sem):
                return pltpu.async_copy(tbl.at[sidx.at[i]], buf, sem)

            def scat(i, buf, sem):
                return pltpu.async_copy(buf, acc.at[didx.at[i]], sem, add=True)

            gather(0, rowsA, gA)

            def body(i, _):
                j = 2 * i
                pltpu.make_async_copy(tbl.at[sidx.at[j]], rowsA, gA).wait()
                scat(j, rowsA, sA)

                @pl.when(i > 0)
                def _():
                    pltpu.make_async_copy(rowsB, acc.at[didx.at[j]], sB).wait()

                gather(j + 1, rowsB, gB)
                pltpu.make_async_copy(tbl.at[sidx.at[j]], rowsB, gB).wait()
                scat(j + 1, rowsB, sB)
                pltpu.make_async_copy(rowsA, acc.at[didx.at[j]], sA).wait()
                gather(j + 2, rowsA, gA)
                return ()

            lax.fori_loop(0, CPW // 2, body, (), unroll=False)
            pltpu.make_async_copy(tbl.at[sidx.at[0]], rowsA, gA).wait()
            pltpu.make_async_copy(rowsB, acc.at[didx.at[0]], sB).wait()
        plsc.subcore_barrier()
        pltpu.sync_copy(acc.at[sl], out_hbm.at[c, sl])

    return agg


_agg = {d: _make_agg(d, False) for d in (16, 32, 64)}
_deg_agg = _make_agg(16, True)


def _tc(fn, *args, out_shape):
    return pl.pallas_call(fn, out_shape=out_shape)(*args)


def _tc0_body(dp, xp, w1, dis_o, t1_o):
    dsum = dp[0] + dp[1] - 1.0
    dis = lax.rsqrt(dsum[:, 0:1])
    dis_o[...] = dis
    t1_o[...] = jnp.dot(xp[...], w1[...], preferred_element_type=jnp.float32) * dis


def _enc_body(u, t, dis, b, w, out):
    v = (u[0] + u[1] - t[...]) * dis[...] + b[...]
    h = jnp.maximum(v, 0.0)
    out[...] = jnp.dot(h, w[...], preferred_element_type=jnp.float32) * dis[...]


def _mid_body(u, t, dis, b, out):
    z = (u[0] + u[1] - t[...]) * dis[...] + b[...]
    out[...] = z * dis[...]


def _dec_body(u, t, dis, b, w, out):
    q = (u[0] + u[1] - t[...]) * dis[...]
    h = jnp.maximum(jnp.dot(q, w[...], preferred_element_type=jnp.float32) + b[...], 0.0)
    out[...] = h * dis[...]


def _fin_body(u, t, dis, b, w, out):
    q = (u[0] + u[1] - t[...]) * dis[...]
    out[...] = jnp.dot(q, w[...], preferred_element_type=jnp.float32) + b[...]


def kernel(x, edge_index, W1, b1, W2, b2, W3, b3, W4, b4, W5, b5, W6, b6):
    f32 = jnp.float32
    pad = EP - E
    src = jnp.concatenate([edge_index[0], jnp.zeros((pad,), jnp.int32)])
    dst = jnp.concatenate([edge_index[1], jnp.full((pad,), NP - 1, jnp.int32)])
    src = src.reshape(NW, CPW, CHUNK)
    src = jnp.concatenate([src, jnp.zeros((NW, 1, CHUNK), jnp.int32)], axis=1)
    dst = dst.reshape(NW, CPW, CHUNK)
    xp = jnp.concatenate([x, jnp.zeros((NP - N, x.shape[1]), f32)])
    ones16 = jnp.concatenate([jnp.ones((N, 16), f32), jnp.zeros((NP - N, 16), f32)])
    b1, b2, b3 = b1.reshape(1, -1), b2.reshape(1, -1), b3.reshape(1, -1)
    b4, b5, b6 = b4.reshape(1, -1), b5.reshape(1, -1), b6.reshape(1, -1)

    def agg(table):
        d = table.shape[-1]
        return _agg[d](table, src, dst)

    dp = _deg_agg(ones16, src, dst)
    dis, t1 = _tc(
        _tc0_body, dp, xp, W1,
        out_shape=(jax.ShapeDtypeStruct((NP, 1), f32),
                   jax.ShapeDtypeStruct((NP, 64), f32)),
    )
    u1 = agg(t1)
    t2 = _tc(_enc_body, u1, t1, dis, b1, W2,
             out_shape=jax.ShapeDtypeStruct((NP, 32), f32))
    u2 = agg(t2)
    t3 = _tc(_enc_body, u2, t2, dis, b2, W3,
             out_shape=jax.ShapeDtypeStruct((NP, 16), f32))
    u3 = agg(t3)
    t4 = _tc(_mid_body, u3, t3, dis, b3,
             out_shape=jax.ShapeDtypeStruct((NP, 16), f32))
    u4 = agg(t4)
    t5 = _tc(_dec_body, u4, t4, dis, b4, W4,
             out_shape=jax.ShapeDtypeStruct((NP, 32), f32))
    u5 = agg(t5)
    t6 = _tc(_dec_body, u5, t5, dis, b5, W5,
             out_shape=jax.ShapeDtypeStruct((NP, 64), f32))
    u6 = agg(t6)
    outp = _tc(_fin_body, u6, t6, dis, b6, W6,
               out_shape=jax.ShapeDtypeStruct((NP, 128), f32))
    return outp[:N]


# speedup vs baseline: 39.1640x; 1.1401x over previous
import functools

import jax
import jax.numpy as jnp
from jax import lax
from jax.experimental import pallas as pl
from jax.experimental.pallas import tpu as pltpu
from jax.experimental.pallas import tpu_sc as plsc

N = 10000
NP = 10240
E = 320000
NC, NS = 2, 16
NW = NC * NS
CH = 128
NCK = E // CH
CPW = NCK // NW
XBASE = NW * CPW
RPN = N // NS
RPP = NP // NS


def _make_agg(D):
    mesh = plsc.VectorSubcoreMesh(
        core_axis_name="c", subcore_axis_name="s", num_cores=NC, num_subcores=NS
    )
    NBUF = 2 if D > 32 else 4
    A = NBUF // 2
    scratch = [
        pltpu.VMEM((CPW + 1, CH), jnp.int32),
        pltpu.VMEM((CPW + 1, CH), jnp.int32),
    ]
    scratch += [pltpu.VMEM((CH, D), jnp.float32) for _ in range(NBUF)]
    scratch += [
        pltpu.VMEM_SHARED((N, D), jnp.float32),
        pltpu.VMEM_SHARED((N, D), jnp.float32),
    ]
    scratch += [pltpu.SemaphoreType.DMA for _ in range(2 * NBUF)]

    @functools.partial(
        pl.kernel,
        out_type=jax.ShapeDtypeStruct((NC, N, D), jnp.float32),
        mesh=mesh,
        scratch_types=scratch,
        compiler_params=pltpu.CompilerParams(use_tc_tiling_on_sc=False),
    )
    def agg(table_hbm, ei_hbm, out_hbm, sidx, didx, *bufs_sems):
        rows = bufs_sems[:NBUF]
        acc, tbl = bufs_sems[NBUF], bufs_sems[NBUF + 1]
        gsem = bufs_sems[NBUF + 2:NBUF + 2 + NBUF]
        ssem = bufs_sems[NBUF + 2 + NBUF:]
        c = lax.axis_index("c")
        s = lax.axis_index("s")
        wid = s * NC + c
        sl = pl.ds(s * RPN, RPN)
        ini = pltpu.async_copy(table_hbm.at[sl], acc.at[sl], gsem[0])
        if NBUF == 2:
            zero = jnp.zeros((16,), jnp.int32)
            for j in range(CH // 16):
                sidx[CPW, pl.ds(16 * j, 16)] = zero
        pltpu.sync_copy(ei_hbm.at[0, pl.ds(wid * CPW, CPW)], sidx.at[pl.ds(0, CPW)])
        pltpu.sync_copy(ei_hbm.at[1, pl.ds(wid * CPW, CPW)], didx.at[pl.ds(0, CPW)])

        @pl.when(wid < NCK - XBASE)
        def _():
            pltpu.sync_copy(ei_hbm.at[0, pl.ds(XBASE + wid, 1)],
                            sidx.at[pl.ds(CPW, 1)])
            pltpu.sync_copy(ei_hbm.at[1, pl.ds(XBASE + wid, 1)],
                            didx.at[pl.ds(CPW, 1)])

        pltpu.sync_copy(table_hbm.at[sl], tbl.at[sl])
        ini.wait()
        plsc.subcore_barrier()

        def wait_g(k, m):
            pltpu.make_async_copy(tbl.at[sidx.at[m]], rows[k], gsem[k]).wait()

        def wait_s(k, m):
            pltpu.make_async_copy(rows[k], acc.at[didx.at[m]], ssem[k]).wait()

        def gather(m, k):
            pltpu.async_copy(tbl.at[sidx.at[m]], rows[k], gsem[k])

        def scat(m, k):
            pltpu.async_copy(rows[k], acc.at[didx.at[m]], ssem[k], add=True)

        for k in range(A):
            gather(k, k)

        def body(i, _):
            j = NBUF * i
            for k in range(NBUF):
                m = j + k
                kn = (k + A) % NBUF
                wait_g(k, m)
                scat(m, k)
                if k < A:
                    @pl.when(i > 0)
                    def _(kn=kn, m=m):
                        wait_s(kn, m)
                else:
                    wait_s(kn, m)
                gather(m + A, kn)
            return ()

        if NBUF == 2:
            lax.fori_loop(0, CPW // NBUF, body, (), unroll=False)
            wait_g(0, 0)

            @pl.when(wid < NCK - XBASE)
            def _():
                scat(CPW, 0)

            wait_s(1, 0)

            @pl.when(wid < NCK - XBASE)
            def _():
                wait_s(0, 0)
        else:
            lax.fori_loop(0, (CPW - 2) // NBUF, body, (), unroll=False)
            wait_g(0, 0)
            scat(CPW - 2, 0)
            wait_g(1, 0)
            scat(CPW - 1, 1)
            wait_s(2, 0)
            wait_s(3, 0)

            @pl.when(wid < NCK - XBASE)
            def _():
                gather(CPW, 2)
                wait_g(2, 0)
                scat(CPW, 2)

            wait_s(0, 0)
            wait_s(1, 0)

            @pl.when(wid < NCK - XBASE)
            def _():
                wait_s(2, 0)

        plsc.subcore_barrier()
        pltpu.sync_copy(acc.at[sl], out_hbm.at[c, sl])

    return agg


_agg = {d: _make_agg(d) for d in (16, 32, 64)}


def _make_deg():
    mesh = plsc.VectorSubcoreMesh(
        core_axis_name="c", subcore_axis_name="s", num_cores=NC, num_subcores=NS
    )

    @functools.partial(
        pl.kernel,
        out_type=jax.ShapeDtypeStruct((NC, NP), jnp.float32),
        mesh=mesh,
        scratch_types=[
            pltpu.VMEM((CPW + 1, CH), jnp.int32),
            pltpu.VMEM((RPP,), jnp.float32),
            pltpu.VMEM((CH,), jnp.float32),
            pltpu.VMEM_SHARED((NP,), jnp.float32),
            pltpu.SemaphoreType.DMA,
            pltpu.SemaphoreType.DMA,
        ],
        compiler_params=pltpu.CompilerParams(use_tc_tiling_on_sc=False),
    )
    def deg(ei_hbm, out_hbm, didx, stripe, ones_c, acc, s0, s1):
        c = lax.axis_index("c")
        s = lax.axis_index("s")
        wid = s * NC + c
        sl = pl.ds(s * RPP, RPP)
        pltpu.sync_copy(ei_hbm.at[1, pl.ds(wid * CPW, CPW)], didx.at[pl.ds(0, CPW)])

        @pl.when(wid < NCK - XBASE)
        def _():
            pltpu.sync_copy(ei_hbm.at[1, pl.ds(XBASE + wid, 1)],
                            didx.at[pl.ds(CPW, 1)])

        one = jnp.ones((16,), jnp.float32)
        for j in range(RPP // 16):
            stripe[pl.ds(16 * j, 16)] = one
        for j in range(CH // 16):
            ones_c[pl.ds(16 * j, 16)] = one
        pltpu.sync_copy(stripe, acc.at[sl])
        plsc.subcore_barrier()

        def scat(m, sem):
            pltpu.async_copy(ones_c, acc.at[didx.at[m]], sem, add=True)

        def wait(m, sem):
            pltpu.make_async_copy(ones_c, acc.at[didx.at[m]], sem).wait()

        scat(0, s0)
        scat(1, s1)

        def body(i, _):
            j = 2 * i
            wait(j, s0)
            scat(j + 2, s0)
            wait(j + 1, s1)
            scat(j + 3, s1)
            return ()

        lax.fori_loop(0, CPW // 2 - 1, body, (), unroll=False)
        wait(0, s0)
        wait(0, s1)

        @pl.when(wid < NCK - XBASE)
        def _():
            scat(CPW, s0)
            wait(0, s0)

        plsc.subcore_barrier()
        pltpu.sync_copy(acc.at[sl], out_hbm.at[c, sl])

    return deg


_deg_agg = _make_deg()


def _tc(fn, *args, out_shape):
    grid = 10
    blk = N // grid

    def spec(sh):
        if len(sh) >= 2 and sh[0] == 2 and sh[1] in (N, NP):
            return pl.BlockSpec((2, blk) + sh[2:],
                                lambda i: (0, i) + (0,) * (len(sh) - 2))
        if sh[0] in (N, NP):
            return pl.BlockSpec((blk,) + sh[1:],
                                lambda i: (i,) + (0,) * (len(sh) - 1))
        return pl.BlockSpec(sh, lambda i: (0,) * len(sh))

    multi = isinstance(out_shape, tuple)
    outs = out_shape if multi else (out_shape,)
    out_specs = tuple(spec(o.shape) for o in outs)
    return pl.pallas_call(
        fn,
        grid=(grid,),
        in_specs=[spec(a.shape) for a in args],
        out_specs=out_specs if multi else out_specs[0],
        out_shape=out_shape,
    )(*args)


def _tc0_body(dp, xp, w1, dis_o, t1_o):
    dsum = dp[0] + dp[1] - 1.0
    dis = lax.rsqrt(dsum)
    dis_o[...] = dis
    t1_o[...] = jnp.dot(xp[...], w1[...], preferred_element_type=jnp.float32) * dis


def _enc_body(u, t, dis, b, w, out):
    v = (u[0] + u[1] - t[...]) * dis[...] + b[...]
    h = jnp.maximum(v, 0.0)
    out[...] = jnp.dot(h, w[...], preferred_element_type=jnp.float32) * dis[...]


def _mid_body(u, t, dis, b, out):
    z = (u[0] + u[1] - t[...]) * dis[...] + b[...]
    out[...] = z * dis[...]


def _dec_body(u, t, dis, b, w, out):
    q = (u[0] + u[1] - t[...]) * dis[...]
    h = jnp.maximum(jnp.dot(q, w[...], preferred_element_type=jnp.float32) + b[...], 0.0)
    out[...] = h * dis[...]


def _fin_body(u, t, dis, b, w, out):
    q = (u[0] + u[1] - t[...]) * dis[...]
    out[...] = jnp.dot(q, w[...], preferred_element_type=jnp.float32) + b[...]


def kernel(x, edge_index, W1, b1, W2, b2, W3, b3, W4, b4, W5, b5, W6, b6):
    f32 = jnp.float32
    ei = edge_index.reshape(2, NCK, CH)
    b1, b2, b3 = b1.reshape(1, -1), b2.reshape(1, -1), b3.reshape(1, -1)
    b4, b5, b6 = b4.reshape(1, -1), b5.reshape(1, -1), b6.reshape(1, -1)

    def agg(table):
        return _agg[table.shape[-1]](table, ei)

    dp = _deg_agg(ei).reshape(2, NP, 1)
    dis, t1 = _tc(
        _tc0_body, dp, x, W1,
        out_shape=(jax.ShapeDtypeStruct((N, 1), f32),
                   jax.ShapeDtypeStruct((N, 64), f32)),
    )
    u1 = agg(t1)
    t2 = _tc(_enc_body, u1, t1, dis, b1, W2,
             out_shape=jax.ShapeDtypeStruct((N, 32), f32))
    u2 = agg(t2)
    t3 = _tc(_enc_body, u2, t2, dis, b2, W3,
             out_shape=jax.ShapeDtypeStruct((N, 16), f32))
    u3 = agg(t3)
    t4 = _tc(_mid_body, u3, t3, dis, b3,
             out_shape=jax.ShapeDtypeStruct((N, 16), f32))
    u4 = agg(t4)
    t5 = _tc(_dec_body, u4, t4, dis, b4, W4,
             out_shape=jax.ShapeDtypeStruct((N, 32), f32))
    u5 = agg(t5)
    t6 = _tc(_dec_body, u5, t5, dis, b5, W5,
             out_shape=jax.ShapeDtypeStruct((N, 64), f32))
    u6 = agg(t6)
    return _tc(_fin_body, u6, t6, dis, b6, W6,
               out_shape=jax.ShapeDtypeStruct((N, 128), f32))


# speedup vs baseline: 39.1718x; 1.0002x over previous
import functools

import jax
import jax.numpy as jnp
from jax import lax
from jax.experimental import pallas as pl
from jax.experimental.pallas import tpu as pltpu
from jax.experimental.pallas import tpu_sc as plsc

N = 10000
NP = 10240
E = 320000
NC, NS = 2, 16
NW = NC * NS
CH = 128
NCK = E // CH
CPW = NCK // NW
XBASE = NW * CPW
RPN = N // NS
RPP = NP // NS


def _make_agg(D):
    mesh = plsc.VectorSubcoreMesh(
        core_axis_name="c", subcore_axis_name="s", num_cores=NC, num_subcores=NS
    )
    NBUF = 2 if D > 32 else 4
    A = NBUF // 2
    scratch = [
        pltpu.VMEM((CPW + 1, CH), jnp.int32),
        pltpu.VMEM((CPW + 1, CH), jnp.int32),
    ]
    scratch += [pltpu.VMEM((CH, D), jnp.float32) for _ in range(NBUF)]
    scratch += [
        pltpu.VMEM_SHARED((N, D), jnp.float32),
        pltpu.VMEM_SHARED((N, D), jnp.float32),
    ]
    scratch += [pltpu.SemaphoreType.DMA for _ in range(2 * NBUF)]

    @functools.partial(
        pl.kernel,
        out_type=jax.ShapeDtypeStruct((NC, N, D), jnp.float32),
        mesh=mesh,
        scratch_types=scratch,
        compiler_params=pltpu.CompilerParams(use_tc_tiling_on_sc=False),
    )
    def agg(table_hbm, ei_hbm, out_hbm, sidx, didx, *bufs_sems):
        rows = bufs_sems[:NBUF]
        acc, tbl = bufs_sems[NBUF], bufs_sems[NBUF + 1]
        gsem = bufs_sems[NBUF + 2:NBUF + 2 + NBUF]
        ssem = bufs_sems[NBUF + 2 + NBUF:]
        c = lax.axis_index("c")
        s = lax.axis_index("s")
        wid = s * NC + c
        sl = pl.ds(s * RPN, RPN)
        ini = pltpu.async_copy(table_hbm.at[sl], acc.at[sl], gsem[0])
        if NBUF == 2:
            zero = jnp.zeros((16,), jnp.int32)
            for j in range(CH // 16):
                sidx[CPW, pl.ds(16 * j, 16)] = zero
        pltpu.sync_copy(ei_hbm.at[0, pl.ds(wid * CPW, CPW)], sidx.at[pl.ds(0, CPW)])
        pltpu.sync_copy(ei_hbm.at[1, pl.ds(wid * CPW, CPW)], didx.at[pl.ds(0, CPW)])

        @pl.when(wid < NCK - XBASE)
        def _():
            pltpu.sync_copy(ei_hbm.at[0, pl.ds(XBASE + wid, 1)],
                            sidx.at[pl.ds(CPW, 1)])
            pltpu.sync_copy(ei_hbm.at[1, pl.ds(XBASE + wid, 1)],
                            didx.at[pl.ds(CPW, 1)])

        pltpu.sync_copy(table_hbm.at[sl], tbl.at[sl])
        ini.wait()
        plsc.subcore_barrier()

        def wait_g(k, m):
            pltpu.make_async_copy(tbl.at[sidx.at[m]], rows[k], gsem[k]).wait()

        def wait_s(k, m):
            pltpu.make_async_copy(rows[k], acc.at[didx.at[m]], ssem[k]).wait()

        def gather(m, k):
            pltpu.async_copy(tbl.at[sidx.at[m]], rows[k], gsem[k])

        def scat(m, k):
            pltpu.async_copy(rows[k], acc.at[didx.at[m]], ssem[k], add=True)

        for k in range(A):
            gather(k, k)

        def body(i, _):
            j = NBUF * i
            for k in range(NBUF):
                m = j + k
                kn = (k + A) % NBUF
                wait_g(k, m)
                scat(m, k)
                if k < A:
                    @pl.when(i > 0)
                    def _(kn=kn, m=m):
                        wait_s(kn, m)
                else:
                    wait_s(kn, m)
                gather(m + A, kn)
            return ()

        if NBUF == 2:
            lax.fori_loop(0, CPW // NBUF, body, (), unroll=False)
            wait_g(0, 0)

            @pl.when(wid < NCK - XBASE)
            def _():
                scat(CPW, 0)

            wait_s(1, 0)

            @pl.when(wid < NCK - XBASE)
            def _():
                wait_s(0, 0)
        else:
            lax.fori_loop(0, (CPW - 2) // NBUF, body, (), unroll=False)
            wait_g(0, 0)
            scat(CPW - 2, 0)
            wait_g(1, 0)
            scat(CPW - 1, 1)
            wait_s(2, 0)
            wait_s(3, 0)

            @pl.when(wid < NCK - XBASE)
            def _():
                gather(CPW, 2)
                wait_g(2, 0)
                scat(CPW, 2)

            wait_s(0, 0)
            wait_s(1, 0)

            @pl.when(wid < NCK - XBASE)
            def _():
                wait_s(2, 0)

        plsc.subcore_barrier()
        pltpu.sync_copy(acc.at[sl], out_hbm.at[c, sl])

    return agg


_agg = {d: _make_agg(d) for d in (16, 32, 64)}


def _make_deg():
    mesh = plsc.VectorSubcoreMesh(
        core_axis_name="c", subcore_axis_name="s", num_cores=NC, num_subcores=NS
    )

    @functools.partial(
        pl.kernel,
        out_type=jax.ShapeDtypeStruct((NC, NP), jnp.float32),
        mesh=mesh,
        scratch_types=[
            pltpu.VMEM((CPW + 1, CH), jnp.int32),
            pltpu.VMEM((RPP,), jnp.float32),
            pltpu.VMEM((CH,), jnp.float32),
            pltpu.VMEM_SHARED((NP,), jnp.float32),
            pltpu.SemaphoreType.DMA,
            pltpu.SemaphoreType.DMA,
        ],
        compiler_params=pltpu.CompilerParams(use_tc_tiling_on_sc=False),
    )
    def deg(ei_hbm, out_hbm, didx, stripe, ones_c, acc, s0, s1):
        c = lax.axis_index("c")
        s = lax.axis_index("s")
        wid = s * NC + c
        sl = pl.ds(s * RPP, RPP)
        pltpu.sync_copy(ei_hbm.at[1, pl.ds(wid * CPW, CPW)], didx.at[pl.ds(0, CPW)])

        @pl.when(wid < NCK - XBASE)
        def _():
            pltpu.sync_copy(ei_hbm.at[1, pl.ds(XBASE + wid, 1)],
                            didx.at[pl.ds(CPW, 1)])

        one = jnp.ones((16,), jnp.float32)
        for j in range(RPP // 16):
            stripe[pl.ds(16 * j, 16)] = one
        for j in range(CH // 16):
            ones_c[pl.ds(16 * j, 16)] = one
        pltpu.sync_copy(stripe, acc.at[sl])
        plsc.subcore_barrier()

        def scat(m, sem):
            pltpu.async_copy(ones_c, acc.at[didx.at[m]], sem, add=True)

        def wait(m, sem):
            pltpu.make_async_copy(ones_c, acc.at[didx.at[m]], sem).wait()

        scat(0, s0)
        scat(1, s1)

        def body(i, _):
            j = 2 * i
            wait(j, s0)
            scat(j + 2, s0)
            wait(j + 1, s1)
            scat(j + 3, s1)
            return ()

        lax.fori_loop(0, CPW // 2 - 1, body, (), unroll=False)
        wait(0, s0)
        wait(0, s1)

        @pl.when(wid < NCK - XBASE)
        def _():
            scat(CPW, s0)
            wait(0, s0)

        plsc.subcore_barrier()
        pltpu.sync_copy(acc.at[sl], out_hbm.at[c, sl])

    return deg


_deg_agg = _make_deg()


def _tc(fn, *args, out_shape, grid=10):
    blk = N // grid

    def spec(sh):
        if len(sh) == 3 and sh[0] == 2 and sh[2] == 128 and sh[1] not in (N, NP):
            return pl.BlockSpec((2, blk * sh[1] // N, 128),
                                lambda i: (0, i, 0))
        if len(sh) >= 2 and sh[0] == 2 and sh[1] in (N, NP):
            return pl.BlockSpec((2, blk) + sh[2:],
                                lambda i: (0, i) + (0,) * (len(sh) - 2))
        if sh[0] in (N, NP):
            return pl.BlockSpec((blk,) + sh[1:],
                                lambda i: (i,) + (0,) * (len(sh) - 1))
        return pl.BlockSpec(sh, lambda i: (0,) * len(sh))

    multi = isinstance(out_shape, tuple)
    outs = out_shape if multi else (out_shape,)
    out_specs = tuple(spec(o.shape) for o in outs)
    return pl.pallas_call(
        fn,
        grid=(grid,),
        in_specs=[spec(a.shape) for a in args],
        out_specs=out_specs if multi else out_specs[0],
        out_shape=out_shape,
    )(*args)


def _tc0_body(dp, xp, w1, dis_o, t1_o):
    dsum = dp[0] + dp[1] - 1.0
    dis = lax.rsqrt(dsum)
    dis_o[...] = dis
    t1_o[...] = jnp.dot(xp[...], w1[...], preferred_element_type=jnp.float32) * dis


def _enc_body(u, t, dis, b, w, out):
    v = (u[0] + u[1] - t[...]) * dis[...] + b[...]
    h = jnp.maximum(v, 0.0)
    out[...] = jnp.dot(h, w[...], preferred_element_type=jnp.float32) * dis[...]


def _mid_body(u, t, dis, b, out):
    z = (u[0] + u[1] - t[...]) * dis[...] + b[...]
    out[...] = z * dis[...]


def _dec_body(u, t, dis, b, w, out):
    q = (u[0] + u[1] - t[...]) * dis[...]
    h = jnp.maximum(jnp.dot(q, w[...], preferred_element_type=jnp.float32) + b[...], 0.0)
    out[...] = h * dis[...]


def _fin_body(u, t, dis, b, w, out):
    q = (u[0] + u[1] - t[...]) * dis[...]
    out[...] = jnp.dot(q, w[...], preferred_element_type=jnp.float32) + b[...]


def kernel(x, edge_index, W1, b1, W2, b2, W3, b3, W4, b4, W5, b5, W6, b6):
    f32 = jnp.float32
    ei = edge_index.reshape(2, NCK, CH)
    b1, b2, b3 = b1.reshape(1, -1), b2.reshape(1, -1), b3.reshape(1, -1)
    b4, b5, b6 = b4.reshape(1, -1), b5.reshape(1, -1), b6.reshape(1, -1)

    def agg(table):
        return _agg[table.shape[-1]](table, ei)

    dp = _deg_agg(ei).reshape(2, NP, 1)
    dis, t1 = _tc(
        _tc0_body, dp, x, W1,
        out_shape=(jax.ShapeDtypeStruct((N, 1), f32),
                   jax.ShapeDtypeStruct((N, 64), f32)),
    )
    u1 = agg(t1)
    t2 = _tc(_enc_body, u1, t1, dis, b1, W2,
             out_shape=jax.ShapeDtypeStruct((N, 32), f32))
    u2 = agg(t2)
    t3 = _tc(_enc_body, u2, t2, dis, b2, W3,
             out_shape=jax.ShapeDtypeStruct((N, 16), f32))
    u3 = agg(t3)
    t4 = _tc(_mid_body, u3, t3, dis, b3,
             out_shape=jax.ShapeDtypeStruct((N, 16), f32))
    u4 = agg(t4)
    t5 = _tc(_dec_body, u4, t4, dis, b4, W4,
             out_shape=jax.ShapeDtypeStruct((N, 32), f32))
    u5 = agg(t5)
    t6 = _tc(_dec_body, u5, t5, dis, b5, W5,
             out_shape=jax.ShapeDtypeStruct((N, 64), f32))
    u6 = agg(t6)
    return _tc(_fin_body, u6, t6, dis, b6, W6,
               out_shape=jax.ShapeDtypeStruct((N, 128), f32))


# speedup vs baseline: 40.3683x; 1.0305x over previous
import functools

import jax
import jax.numpy as jnp
from jax import lax
from jax.experimental import pallas as pl
from jax.experimental.pallas import tpu as pltpu
from jax.experimental.pallas import tpu_sc as plsc

N = 10000
NP = 10240
E = 320000
NC, NS = 2, 16
NW = NC * NS
CH = 128
NCK = E // CH
CPW = NCK // NW
XBASE = NW * CPW
RPN = N // NS
RPP = NP // NS


def _make_agg(D):
    mesh = plsc.VectorSubcoreMesh(
        core_axis_name="c", subcore_axis_name="s", num_cores=NC, num_subcores=NS
    )
    NBUF = 2 if D > 32 else 4
    A = NBUF // 2
    scratch = [
        pltpu.VMEM((CPW + 1, CH), jnp.int32),
        pltpu.VMEM((CPW + 1, CH), jnp.int32),
    ]
    scratch += [pltpu.VMEM((CH, D), jnp.float32) for _ in range(NBUF)]
    scratch += [
        pltpu.VMEM_SHARED((N, D), jnp.float32),
        pltpu.VMEM_SHARED((N, D), jnp.float32),
    ]
    scratch += [pltpu.SemaphoreType.DMA for _ in range(2 * NBUF)]

    @functools.partial(
        pl.kernel,
        out_type=jax.ShapeDtypeStruct((NC, N, D), jnp.float32),
        mesh=mesh,
        scratch_types=scratch,
        compiler_params=pltpu.CompilerParams(use_tc_tiling_on_sc=False),
    )
    def agg(table_hbm, ei_hbm, out_hbm, sidx, didx, *bufs_sems):
        rows = bufs_sems[:NBUF]
        acc, tbl = bufs_sems[NBUF], bufs_sems[NBUF + 1]
        gsem = bufs_sems[NBUF + 2:NBUF + 2 + NBUF]
        ssem = bufs_sems[NBUF + 2 + NBUF:]
        c = lax.axis_index("c")
        s = lax.axis_index("s")
        wid = s * NC + c
        sl = pl.ds(s * RPN, RPN)
        ini = pltpu.async_copy(table_hbm.at[sl], acc.at[sl], gsem[0])
        if NBUF == 2:
            zero = jnp.zeros((16,), jnp.int32)
            for j in range(CH // 16):
                sidx[CPW, pl.ds(16 * j, 16)] = zero
        pltpu.sync_copy(ei_hbm.at[0, pl.ds(wid * CPW, CPW)], sidx.at[pl.ds(0, CPW)])
        pltpu.sync_copy(ei_hbm.at[1, pl.ds(wid * CPW, CPW)], didx.at[pl.ds(0, CPW)])

        @pl.when(wid < NCK - XBASE)
        def _():
            pltpu.sync_copy(ei_hbm.at[0, pl.ds(XBASE + wid, 1)],
                            sidx.at[pl.ds(CPW, 1)])
            pltpu.sync_copy(ei_hbm.at[1, pl.ds(XBASE + wid, 1)],
                            didx.at[pl.ds(CPW, 1)])

        pltpu.sync_copy(table_hbm.at[sl], tbl.at[sl])
        ini.wait()
        plsc.subcore_barrier()

        def wait_g(k, m):
            pltpu.make_async_copy(tbl.at[sidx.at[m]], rows[k], gsem[k]).wait()

        def wait_s(k, m):
            pltpu.make_async_copy(rows[k], acc.at[didx.at[m]], ssem[k]).wait()

        def gather(m, k):
            pltpu.async_copy(tbl.at[sidx.at[m]], rows[k], gsem[k])

        def scat(m, k):
            pltpu.async_copy(rows[k], acc.at[didx.at[m]], ssem[k], add=True)

        for k in range(A):
            gather(k, k)

        def body(i, _):
            j = NBUF * i
            for k in range(NBUF):
                m = j + k
                kn = (k + A) % NBUF
                wait_g(k, m)
                scat(m, k)
                if k < A:
                    @pl.when(i > 0)
                    def _(kn=kn, m=m):
                        wait_s(kn, m)
                else:
                    wait_s(kn, m)
                gather(m + A, kn)
            return ()

        if NBUF == 2:
            lax.fori_loop(0, CPW // NBUF, body, (), unroll=False)
            wait_g(0, 0)

            @pl.when(wid < NCK - XBASE)
            def _():
                scat(CPW, 0)

            wait_s(1, 0)

            @pl.when(wid < NCK - XBASE)
            def _():
                wait_s(0, 0)
        else:
            lax.fori_loop(0, (CPW - 2) // NBUF, body, (), unroll=False)
            wait_g(0, 0)
            scat(CPW - 2, 0)
            wait_g(1, 0)
            scat(CPW - 1, 1)
            wait_s(2, 0)
            wait_s(3, 0)

            @pl.when(wid < NCK - XBASE)
            def _():
                gather(CPW, 2)
                wait_g(2, 0)
                scat(CPW, 2)

            wait_s(0, 0)
            wait_s(1, 0)

            @pl.when(wid < NCK - XBASE)
            def _():
                wait_s(2, 0)

        plsc.subcore_barrier()
        pltpu.sync_copy(acc.at[sl], out_hbm.at[c, sl])

    return agg


_agg = {d: _make_agg(d) for d in (16, 32, 64)}


def _make_deg():
    mesh = plsc.VectorSubcoreMesh(
        core_axis_name="c", subcore_axis_name="s", num_cores=NC, num_subcores=NS
    )

    @functools.partial(
        pl.kernel,
        out_type=jax.ShapeDtypeStruct((NC, NP), jnp.float32),
        mesh=mesh,
        scratch_types=[
            pltpu.VMEM((CPW + 1, CH), jnp.int32),
            pltpu.VMEM((RPP,), jnp.float32),
            pltpu.VMEM((CH,), jnp.float32),
            pltpu.VMEM_SHARED((NP,), jnp.float32),
            pltpu.SemaphoreType.DMA,
            pltpu.SemaphoreType.DMA,
        ],
        compiler_params=pltpu.CompilerParams(use_tc_tiling_on_sc=False),
    )
    def deg(ei_hbm, out_hbm, didx, stripe, ones_c, acc, s0, s1):
        c = lax.axis_index("c")
        s = lax.axis_index("s")
        wid = s * NC + c
        sl = pl.ds(s * RPP, RPP)
        pltpu.sync_copy(ei_hbm.at[1, pl.ds(wid * CPW, CPW)], didx.at[pl.ds(0, CPW)])

        @pl.when(wid < NCK - XBASE)
        def _():
            pltpu.sync_copy(ei_hbm.at[1, pl.ds(XBASE + wid, 1)],
                            didx.at[pl.ds(CPW, 1)])

        one = jnp.ones((16,), jnp.float32)
        for j in range(RPP // 16):
            stripe[pl.ds(16 * j, 16)] = one
        for j in range(CH // 16):
            ones_c[pl.ds(16 * j, 16)] = one
        pltpu.sync_copy(stripe, acc.at[sl])
        plsc.subcore_barrier()

        def scat(m, sem):
            pltpu.async_copy(ones_c, acc.at[didx.at[m]], sem, add=True)

        def wait(m, sem):
            pltpu.make_async_copy(ones_c, acc.at[didx.at[m]], sem).wait()

        scat(0, s0)
        scat(1, s1)

        def body(i, _):
            j = 2 * i
            wait(j, s0)
            scat(j + 2, s0)
            wait(j + 1, s1)
            scat(j + 3, s1)
            return ()

        lax.fori_loop(0, CPW // 2 - 1, body, (), unroll=False)
        wait(0, s0)
        wait(0, s1)

        @pl.when(wid < NCK - XBASE)
        def _():
            scat(CPW, s0)
            wait(0, s0)

        plsc.subcore_barrier()
        pltpu.sync_copy(acc.at[sl], out_hbm.at[c, sl])

    return deg


_deg_agg = _make_deg()


def _tc(fn, *args, out_shape, grid=10):
    blk = N // grid

    def spec(sh):
        if len(sh) == 3 and sh[0] == 2 and sh[2] == 128 and sh[1] not in (N, NP):
            return pl.BlockSpec((2, blk * sh[1] // N, 128),
                                lambda i: (0, i, 0))
        if len(sh) >= 2 and sh[0] == 2 and sh[1] in (N, NP):
            return pl.BlockSpec((2, blk) + sh[2:],
                                lambda i: (0, i) + (0,) * (len(sh) - 2))
        if sh[0] in (N, NP):
            return pl.BlockSpec((blk,) + sh[1:],
                                lambda i: (i,) + (0,) * (len(sh) - 1))
        return pl.BlockSpec(sh, lambda i: (0,) * len(sh))

    multi = isinstance(out_shape, tuple)
    outs = out_shape if multi else (out_shape,)
    out_specs = tuple(spec(o.shape) for o in outs)
    return pl.pallas_call(
        fn,
        grid=(grid,),
        in_specs=[spec(a.shape) for a in args],
        out_specs=out_specs if multi else out_specs[0],
        out_shape=out_shape,
    )(*args)


def _tc0_body(dp, xp, w1, dis_o, t1_o):
    dsum = dp[0, :N] + dp[1, :N] - 1.0
    dis = lax.rsqrt(dsum)[:, None]
    dis_o[...] = dis
    t1_o[...] = jnp.dot(xp[...], w1[...], preferred_element_type=jnp.float32) * dis


def _enc_body(u, t, dis, b, w, out):
    v = (u[0] + u[1] - t[...]) * dis[...] + b[...]
    h = jnp.maximum(v, 0.0)
    out[...] = jnp.dot(h, w[...], preferred_element_type=jnp.float32) * dis[...]


def _mid_body(u, t, dis, b, out):
    z = (u[0] + u[1] - t[...]) * dis[...] + b[...]
    out[...] = z * dis[...]


def _dec_body(u, t, dis, b, w, out):
    q = (u[0] + u[1] - t[...]) * dis[...]
    h = jnp.maximum(jnp.dot(q, w[...], preferred_element_type=jnp.float32) + b[...], 0.0)
    out[...] = h * dis[...]


def _fin_body(u, t, dis, b, w, out):
    q = (u[0] + u[1] - t[...]) * dis[...]
    out[...] = jnp.dot(q, w[...], preferred_element_type=jnp.float32) + b[...]


def kernel(x, edge_index, W1, b1, W2, b2, W3, b3, W4, b4, W5, b5, W6, b6):
    f32 = jnp.float32
    ei = edge_index.reshape(2, NCK, CH)
    b1, b2, b3 = b1.reshape(1, -1), b2.reshape(1, -1), b3.reshape(1, -1)
    b4, b5, b6 = b4.reshape(1, -1), b5.reshape(1, -1), b6.reshape(1, -1)

    def agg(table):
        return _agg[table.shape[-1]](table, ei)

    dp = _deg_agg(ei)
    dis, t1 = pl.pallas_call(
        _tc0_body,
        out_shape=(jax.ShapeDtypeStruct((N, 1), f32),
                   jax.ShapeDtypeStruct((N, 64), f32)),
    )(dp, x, W1)
    u1 = agg(t1)
    t2 = _tc(_enc_body, u1, t1, dis, b1, W2,
             out_shape=jax.ShapeDtypeStruct((N, 32), f32))
    u2 = agg(t2)
    t3 = _tc(_enc_body, u2, t2, dis, b2, W3,
             out_shape=jax.ShapeDtypeStruct((N, 16), f32))
    u3 = agg(t3)
    t4 = _tc(_mid_body, u3, t3, dis, b3,
             out_shape=jax.ShapeDtypeStruct((N, 16), f32))
    u4 = agg(t4)
    t5 = _tc(_dec_body, u4, t4, dis, b4, W4,
             out_shape=jax.ShapeDtypeStruct((N, 32), f32))
    u5 = agg(t5)
    t6 = _tc(_dec_body, u5, t5, dis, b5, W5,
             out_shape=jax.ShapeDtypeStruct((N, 64), f32))
    u6 = agg(t6)
    return _tc(_fin_body, u6, t6, dis, b6, W6,
               out_shape=jax.ShapeDtypeStruct((N, 128), f32))


# speedup vs baseline: 53.1559x; 1.3168x over previous
import functools

import jax
import jax.numpy as jnp
from jax import lax
from jax.experimental import pallas as pl
from jax.experimental.pallas import tpu as pltpu
from jax.experimental.pallas import tpu_sc as plsc

N = 10000
NP = 10240
E = 320000
NC, NS = 2, 16
NW = NC * NS
CH = 128
NCK = E // CH
CPW = NCK // NW
XBASE = NW * CPW
RPN = N // NS
RPP = NP // NS


def _make_agg(D):
    mesh = plsc.VectorSubcoreMesh(
        core_axis_name="c", subcore_axis_name="s", num_cores=NC, num_subcores=NS
    )
    NBUF = 2 if D > 32 else 4
    A = NBUF // 2
    scratch = [
        pltpu.VMEM((CPW + 1, CH), jnp.int32),
        pltpu.VMEM((CPW + 1, CH), jnp.int32),
    ]
    scratch += [pltpu.VMEM((CH, D), jnp.float32) for _ in range(NBUF)]
    scratch += [
        pltpu.VMEM_SHARED((N, D), jnp.float32),
        pltpu.VMEM_SHARED((N, D), jnp.float32),
    ]
    scratch += [pltpu.SemaphoreType.DMA for _ in range(2 * NBUF)]

    @functools.partial(
        pl.kernel,
        out_type=jax.ShapeDtypeStruct((NC, N, D), jnp.float32),
        mesh=mesh,
        scratch_types=scratch,
        compiler_params=pltpu.CompilerParams(use_tc_tiling_on_sc=False),
    )
    def agg(table_hbm, ei_hbm, out_hbm, sidx, didx, *bufs_sems):
        rows = bufs_sems[:NBUF]
        acc, tbl = bufs_sems[NBUF], bufs_sems[NBUF + 1]
        gsem = bufs_sems[NBUF + 2:NBUF + 2 + NBUF]
        ssem = bufs_sems[NBUF + 2 + NBUF:]
        c = lax.axis_index("c")
        s = lax.axis_index("s")
        wid = s * NC + c
        sl = pl.ds(s * RPN, RPN)
        ini = pltpu.async_copy(table_hbm.at[sl], acc.at[sl], gsem[0])
        if NBUF == 2:
            zero = jnp.zeros((16,), jnp.int32)
            for j in range(CH // 16):
                sidx[CPW, pl.ds(16 * j, 16)] = zero
        pltpu.sync_copy(ei_hbm.at[0, pl.ds(wid * CPW, CPW)], sidx.at[pl.ds(0, CPW)])
        pltpu.sync_copy(ei_hbm.at[1, pl.ds(wid * CPW, CPW)], didx.at[pl.ds(0, CPW)])

        @pl.when(wid < NCK - XBASE)
        def _():
            pltpu.sync_copy(ei_hbm.at[0, pl.ds(XBASE + wid, 1)],
                            sidx.at[pl.ds(CPW, 1)])
            pltpu.sync_copy(ei_hbm.at[1, pl.ds(XBASE + wid, 1)],
                            didx.at[pl.ds(CPW, 1)])

        pltpu.sync_copy(table_hbm.at[sl], tbl.at[sl])
        ini.wait()
        plsc.subcore_barrier()

        def wait_g(k, m):
            pltpu.make_async_copy(tbl.at[sidx.at[m]], rows[k], gsem[k]).wait()

        def wait_s(k, m):
            pltpu.make_async_copy(rows[k], acc.at[didx.at[m]], ssem[k]).wait()

        def gather(m, k):
            pltpu.async_copy(tbl.at[sidx.at[m]], rows[k], gsem[k])

        def scat(m, k):
            pltpu.async_copy(rows[k], acc.at[didx.at[m]], ssem[k], add=True)

        for k in range(A):
            gather(k, k)

        def body(i, _):
            j = NBUF * i
            for k in range(NBUF):
                m = j + k
                kn = (k + A) % NBUF
                wait_g(k, m)
                scat(m, k)
                if k < A:
                    @pl.when(i > 0)
                    def _(kn=kn, m=m):
                        wait_s(kn, m)
                else:
                    wait_s(kn, m)
                gather(m + A, kn)
            return ()

        if NBUF == 2:
            lax.fori_loop(0, CPW // NBUF, body, (), unroll=False)
            wait_g(0, 0)

            @pl.when(wid < NCK - XBASE)
            def _():
                scat(CPW, 0)

            wait_s(1, 0)

            @pl.when(wid < NCK - XBASE)
            def _():
                wait_s(0, 0)
        else:
            lax.fori_loop(0, (CPW - 2) // NBUF, body, (), unroll=False)
            wait_g(0, 0)
            scat(CPW - 2, 0)
            wait_g(1, 0)
            scat(CPW - 1, 1)
            wait_s(2, 0)
            wait_s(3, 0)

            @pl.when(wid < NCK - XBASE)
            def _():
                gather(CPW, 2)
                wait_g(2, 0)
                scat(CPW, 2)

            wait_s(0, 0)
            wait_s(1, 0)

            @pl.when(wid < NCK - XBASE)
            def _():
                wait_s(2, 0)

        plsc.subcore_barrier()
        pltpu.sync_copy(acc.at[sl], out_hbm.at[c, sl])

    return agg


_agg = {d: _make_agg(d) for d in (16, 32, 64)}


def _make_deg():
    mesh = plsc.VectorSubcoreMesh(
        core_axis_name="c", subcore_axis_name="s", num_cores=NC, num_subcores=NS
    )

    @functools.partial(
        pl.kernel,
        out_type=jax.ShapeDtypeStruct((NC, NP), jnp.float32),
        mesh=mesh,
        scratch_types=[
            pltpu.VMEM((CPW + 1, CH), jnp.int32),
            pltpu.VMEM((RPP,), jnp.float32),
            pltpu.VMEM((CH,), jnp.float32),
            pltpu.VMEM_SHARED((NP,), jnp.float32),
            pltpu.SemaphoreType.DMA,
            pltpu.SemaphoreType.DMA,
        ],
        compiler_params=pltpu.CompilerParams(use_tc_tiling_on_sc=False),
    )
    def deg(ei_hbm, out_hbm, didx, stripe, ones_c, acc, s0, s1):
        c = lax.axis_index("c")
        s = lax.axis_index("s")
        wid = s * NC + c
        sl = pl.ds(s * RPP, RPP)
        pltpu.sync_copy(ei_hbm.at[1, pl.ds(wid * CPW, CPW)], didx.at[pl.ds(0, CPW)])

        @pl.when(wid < NCK - XBASE)
        def _():
            pltpu.sync_copy(ei_hbm.at[1, pl.ds(XBASE + wid, 1)],
                            didx.at[pl.ds(CPW, 1)])

        one = jnp.ones((16,), jnp.float32)
        for j in range(RPP // 16):
            stripe[pl.ds(16 * j, 16)] = one
        for j in range(CH // 16):
            ones_c[pl.ds(16 * j, 16)] = one
        pltpu.sync_copy(stripe, acc.at[sl])
        plsc.subcore_barrier()

        def scat(m, sem):
            pltpu.async_copy(ones_c, acc.at[didx.at[m]], sem, add=True)

        def wait(m, sem):
            pltpu.make_async_copy(ones_c, acc.at[didx.at[m]], sem).wait()

        scat(0, s0)
        scat(1, s1)

        def body(i, _):
            j = 2 * i
            wait(j, s0)
            scat(j + 2, s0)
            wait(j + 1, s1)
            scat(j + 3, s1)
            return ()

        lax.fori_loop(0, CPW // 2 - 1, body, (), unroll=False)
        wait(0, s0)
        wait(0, s1)

        @pl.when(wid < NCK - XBASE)
        def _():
            scat(CPW, s0)
            wait(0, s0)

        plsc.subcore_barrier()
        pltpu.sync_copy(acc.at[sl], out_hbm.at[c, sl])

    return deg


_deg_agg = _make_deg()


def _tc(fn, *args, out_shape, grid=10):
    blk = N // grid

    def spec(sh):
        if len(sh) == 3 and sh[0] == 2 and sh[2] == 128 and sh[1] not in (N, NP):
            return pl.BlockSpec((2, blk * sh[1] // N, 128),
                                lambda i: (0, i, 0))
        if len(sh) >= 2 and sh[0] == 2 and sh[1] in (N, NP):
            return pl.BlockSpec((2, blk) + sh[2:],
                                lambda i: (0, i) + (0,) * (len(sh) - 2))
        if sh[0] in (N, NP):
            return pl.BlockSpec((blk,) + sh[1:],
                                lambda i: (i,) + (0,) * (len(sh) - 1))
        return pl.BlockSpec(sh, lambda i: (0,) * len(sh))

    multi = isinstance(out_shape, tuple)
    outs = out_shape if multi else (out_shape,)
    out_specs = tuple(spec(o.shape) for o in outs)
    return pl.pallas_call(
        fn,
        grid=(grid,),
        in_specs=[spec(a.shape) for a in args],
        out_specs=out_specs if multi else out_specs[0],
        out_shape=out_shape,
        compiler_params=pltpu.CompilerParams(
            allow_input_fusion=[True] * len(args)),
    )(*args)


def _tc0_body(dp, xp, w1, dis_o, t1_o):
    dsum = dp[0, :N] + dp[1, :N] - 1.0
    dis = lax.rsqrt(dsum)[:, None]
    dis_o[...] = dis
    t1_o[...] = jnp.dot(xp[...], w1[...], preferred_element_type=jnp.float32) * dis


def _enc_body(u, t, dis, b, w, out):
    v = (u[0] + u[1] - t[...]) * dis[...] + b[...]
    h = jnp.maximum(v, 0.0)
    out[...] = jnp.dot(h, w[...], preferred_element_type=jnp.float32) * dis[...]


def _mid_body(u, t, dis, b, out):
    z = (u[0] + u[1] - t[...]) * dis[...] + b[...]
    out[...] = z * dis[...]


def _dec_body(u, t, dis, b, w, out):
    q = (u[0] + u[1] - t[...]) * dis[...]
    h = jnp.maximum(jnp.dot(q, w[...], preferred_element_type=jnp.float32) + b[...], 0.0)
    out[...] = h * dis[...]


def _fin_body(u, t, dis, b, w, out):
    q = (u[0] + u[1] - t[...]) * dis[...]
    out[...] = jnp.dot(q, w[...], preferred_element_type=jnp.float32) + b[...]


def kernel(x, edge_index, W1, b1, W2, b2, W3, b3, W4, b4, W5, b5, W6, b6):
    f32 = jnp.float32
    ei = edge_index.reshape(2, NCK, CH)
    b1, b2, b3 = b1.reshape(1, -1), b2.reshape(1, -1), b3.reshape(1, -1)
    b4, b5, b6 = b4.reshape(1, -1), b5.reshape(1, -1), b6.reshape(1, -1)

    def agg(table):
        return _agg[table.shape[-1]](table, ei)

    dp = _deg_agg(ei)
    dis, t1 = pl.pallas_call(
        _tc0_body,
        out_shape=(jax.ShapeDtypeStruct((N, 1), f32),
                   jax.ShapeDtypeStruct((N, 64), f32)),
    )(dp, x, W1)
    u1 = agg(t1)
    t2 = _tc(_enc_body, u1, t1, dis, b1, W2,
             out_shape=jax.ShapeDtypeStruct((N, 32), f32))
    u2 = agg(t2)
    t3 = _tc(_enc_body, u2, t2, dis, b2, W3,
             out_shape=jax.ShapeDtypeStruct((N, 16), f32))
    u3 = agg(t3)
    t4 = _tc(_mid_body, u3, t3, dis, b3,
             out_shape=jax.ShapeDtypeStruct((N, 16), f32))
    u4 = agg(t4)
    t5 = _tc(_dec_body, u4, t4, dis, b4, W4,
             out_shape=jax.ShapeDtypeStruct((N, 32), f32))
    u5 = agg(t5)
    t6 = _tc(_dec_body, u5, t5, dis, b5, W5,
             out_shape=jax.ShapeDtypeStruct((N, 64), f32))
    u6 = agg(t6)
    return _tc(_fin_body, u6, t6, dis, b6, W6,
               out_shape=jax.ShapeDtypeStruct((N, 128), f32))
